# baseline (speedup 1.0000x reference)
import functools
import math

import jax
import jax.numpy as jnp
from jax import lax
from jax.experimental import pallas as pl
from jax.experimental.pallas import tpu as pltpu
from jax.experimental.pallas import tpu_sc as plsc

_B = 32
_SEQ = 2500
_D = 1024
_LANES = 16
_SCALE = math.sqrt(_D)

_NC, _NS = 2, 16
_NW = _NC * _NS


def _sc_body(pos_hbm, x_hbm, pe_hbm, alpha_hbm, out_hbm,
             pos_v, idx_v, row_v, x_v, alpha_v, sem, sem2):
    wid = lax.axis_index("s") * _NC + lax.axis_index("c")
    xcp = pltpu.async_copy(x_hbm.at[wid], x_v, sem2)
    acp = pltpu.async_copy(alpha_hbm, alpha_v, sem2)
    pltpu.sync_copy(pos_hbm, pos_v)
    half = wid // _LANES
    lane = wid % _LANES
    posv = pos_v[pl.ds(pl.multiple_of(half * _LANES, _LANES), _LANES)]
    batchv = half * _LANES + jnp.arange(_LANES, dtype=jnp.int32)
    flatv = (posv - 1) * _B + batchv
    plsc.store_scatter(idx_v, [jnp.arange(_LANES, dtype=jnp.int32) * 8], flatv)
    idx1 = idx_v.at[pl.ds(pl.multiple_of(lane * 8, 8), 1)]
    _H = _D // 2
    g0 = pltpu.async_copy(
        pe_hbm.at[:, pl.ds(0, _H)].at[idx1], row_v.at[:, pl.ds(0, _H)], sem)
    g1 = pltpu.async_copy(
        pe_hbm.at[:, pl.ds(_H, _H)].at[idx1], row_v.at[:, pl.ds(_H, _H)], sem)
    acp.wait()
    xcp.wait()
    alpha_vec = plsc.load_gather(alpha_v, [jnp.zeros((_LANES,), jnp.int32)])

    def fma(j, carry):
        sl = pl.ds(pl.multiple_of(j * _LANES, _LANES), _LANES)
        x_v[0, sl] = x_v[0, sl] * _SCALE + alpha_vec * row_v[0, sl]
        return carry

    g0.wait()
    lax.fori_loop(0, _H // _LANES, fma, 0)
    o0 = pltpu.async_copy(
        x_v.at[:, pl.ds(0, _H)], out_hbm.at[wid].at[:, pl.ds(0, _H)], sem2)
    g1.wait()
    lax.fori_loop(_H // _LANES, _D // _LANES, fma, 0)
    o1 = pltpu.async_copy(
        x_v.at[:, pl.ds(_H, _H)], out_hbm.at[wid].at[:, pl.ds(_H, _H)], sem2)
    o0.wait()
    o1.wait()


_sc_call = functools.partial(
    pl.kernel,
    mesh=plsc.VectorSubcoreMesh(core_axis_name="c", subcore_axis_name="s"),
    compiler_params=pltpu.CompilerParams(needs_layout_passes=False),
    out_type=jax.ShapeDtypeStruct((_B, 1, _D), jnp.float32),
    scratch_types=[
        pltpu.VMEM((_B,), jnp.int32),
        pltpu.VMEM((_LANES * 8,), jnp.int32),
        pltpu.VMEM((1, _D), jnp.float32),
        pltpu.VMEM((1, _D), jnp.float32),
        pltpu.VMEM((1,), jnp.float32),
        pltpu.SemaphoreType.DMA,
        pltpu.SemaphoreType.DMA,
    ],
)(_sc_body)


@jax.jit
def kernel(input_pos, x, pe, alpha):
    pe_rows = pe.transpose(1, 0, 2).reshape(_SEQ * _B, _D)
    return _sc_call(input_pos, x, pe_rows, alpha.astype(jnp.float32))


# speedup vs baseline: 1.0109x; 1.0109x over previous
import functools
import math

import jax
import jax.numpy as jnp
from jax import lax
from jax.experimental import pallas as pl
from jax.experimental.pallas import tpu as pltpu
from jax.experimental.pallas import tpu_sc as plsc

_B = 32
_SEQ = 2500
_D = 1024
_LANES = 16
_SCALE = math.sqrt(_D)

_NC, _NS = 2, 16
_NW = _NC * _NS


def _sc_body(pos_hbm, x_hbm, pe_hbm, alpha_hbm, out_hbm,
             pos_v, idx_v, row_v, x_v, alpha_v, sem, sem2):
    wid = lax.axis_index("s") * _NC + lax.axis_index("c")
    xcp = pltpu.async_copy(x_hbm.at[wid], x_v, sem2)
    acp = pltpu.async_copy(alpha_hbm, alpha_v, sem2)
    pltpu.sync_copy(pos_hbm, pos_v)
    half = wid // _LANES
    lane = wid % _LANES
    posv = pos_v[pl.ds(pl.multiple_of(half * _LANES, _LANES), _LANES)]
    batchv = half * _LANES + jnp.arange(_LANES, dtype=jnp.int32)
    flatv = (posv - 1) * _B + batchv
    plsc.store_scatter(idx_v, [jnp.arange(_LANES, dtype=jnp.int32) * 8], flatv)
    gather = pltpu.async_copy(
        pe_hbm.at[idx_v.at[pl.ds(pl.multiple_of(lane * 8, 8), 1)]], row_v, sem)
    acp.wait()
    xcp.wait()
    gather.wait()
    alpha_vec = plsc.load_gather(alpha_v, [jnp.zeros((_LANES,), jnp.int32)])

    def fma(j, carry):
        sl = pl.ds(pl.multiple_of(j * _LANES, _LANES), _LANES)
        x_v[0, sl] = x_v[0, sl] * _SCALE + alpha_vec * row_v[0, sl]
        return carry

    lax.fori_loop(0, _D // _LANES, fma, 0)
    pltpu.sync_copy(x_v, out_hbm.at[wid])


_sc_call = functools.partial(
    pl.kernel,
    mesh=plsc.VectorSubcoreMesh(core_axis_name="c", subcore_axis_name="s"),
    compiler_params=pltpu.CompilerParams(needs_layout_passes=False),
    out_type=jax.ShapeDtypeStruct((_B, 1, _D), jnp.float32),
    scratch_types=[
        pltpu.VMEM((_B,), jnp.int32),
        pltpu.VMEM((_LANES * 8,), jnp.int32),
        pltpu.VMEM((1, _D), jnp.float32),
        pltpu.VMEM((1, _D), jnp.float32),
        pltpu.VMEM((1,), jnp.float32),
        pltpu.SemaphoreType.DMA,
        pltpu.SemaphoreType.DMA,
    ],
)(_sc_body)


@jax.jit
def kernel(input_pos, x, pe, alpha):
    pe_rows = pe.transpose(1, 0, 2).reshape(_SEQ * _B, _D)
    return _sc_call(input_pos, x, pe_rows, alpha.astype(jnp.float32))
